# baseline (speedup 1.0000x reference)
import functools

import jax
import jax.numpy as jnp
from jax import lax
from jax.experimental import pallas as pl
from jax.experimental.pallas import tpu as pltpu
from jax.experimental.pallas import tpu_sc as plsc

NC = 2
NS = 16
NW = NC * NS
CHUNK = 128
N_PAD = 10240


def _sc_mesh():
    return plsc.VectorSubcoreMesh(core_axis_name="c", subcore_axis_name="s")


def _deg_partials(dst3, ones_rows, zslab):
    nchunks = dst3.shape[1]
    W = ones_rows.shape[1]
    rp = N_PAD // NS

    @functools.partial(
        pl.kernel,
        out_type=jax.ShapeDtypeStruct((NC, N_PAD, W), jnp.float32),
        mesh=_sc_mesh(),
        scratch_types=[
            pltpu.VMEM((nchunks, CHUNK), jnp.int32),
            pltpu.VMEM((CHUNK, W), jnp.float32),
            pltpu.VMEM_SHARED((N_PAD, W), jnp.float32),
        ],
    )
    def k(dst_hbm, ones_hbm, z_hbm, out_hbm, dst_v, ones_v, acc):
        c = lax.axis_index("c")
        s = lax.axis_index("s")
        w = c * NS + s
        pltpu.sync_copy(z_hbm, acc.at[pl.ds(s * rp, rp)])
        pltpu.sync_copy(dst_hbm.at[w], dst_v)
        pltpu.sync_copy(ones_hbm, ones_v)
        plsc.subcore_barrier()

        def body(j, carry):
            pltpu.sync_copy(ones_v, acc.at[dst_v.at[j]], add=True)
            return carry

        lax.fori_loop(0, nchunks, body, 0)
        plsc.subcore_barrier()
        pltpu.sync_copy(acc.at[pl.ds(s * rp, rp)],
                        out_hbm.at[c, pl.ds(s * rp, rp), :])

    return k(dst3, ones_rows, zslab)


def _agg_partials(zs, src3, dst3, zslab):
    nchunks = src3.shape[1]
    C = zs.shape[1]
    rp = N_PAD // NS

    @functools.partial(
        pl.kernel,
        out_type=jax.ShapeDtypeStruct((NC, N_PAD, C), jnp.float32),
        mesh=_sc_mesh(),
        scratch_types=[
            pltpu.VMEM((nchunks, CHUNK), jnp.int32),
            pltpu.VMEM((nchunks, CHUNK), jnp.int32),
            pltpu.VMEM((CHUNK, C), jnp.float32),
            pltpu.VMEM_SHARED((N_PAD, C), jnp.float32),
            pltpu.SemaphoreType.DMA,
        ],
    )
    def k(zs_hbm, src_hbm, dst_hbm, z_hbm, out_hbm,
          src_v, dst_v, rows_v, acc, sem):
        c = lax.axis_index("c")
        s = lax.axis_index("s")
        w = c * NS + s
        pltpu.sync_copy(z_hbm, acc.at[pl.ds(s * rp, rp)])
        pltpu.sync_copy(src_hbm.at[w], src_v)
        pltpu.sync_copy(dst_hbm.at[w], dst_v)
        plsc.subcore_barrier()

        def body(j, carry):
            pltpu.async_copy(zs_hbm.at[src_v.at[j]], rows_v, sem).wait()
            pltpu.sync_copy(rows_v, acc.at[dst_v.at[j]], add=True)
            return carry

        lax.fori_loop(0, nchunks, body, 0)
        plsc.subcore_barrier()
        pltpu.sync_copy(acc.at[pl.ds(s * rp, rp)],
                        out_hbm.at[c, pl.ds(s * rp, rp), :])

    return k(zs, src3, dst3, zslab)


def _dis_block(degp):
    d = degp[0][:, 0:1] + degp[1][:, 0:1] + 1.0
    return lax.rsqrt(d)


def _tc_in_scale(x_pad, W, degp, bm=1024):
    M, K = x_pad.shape
    Cout = W.shape[1]

    def body(x_ref, w_ref, degp_ref, o_ref):
        dis = _dis_block(degp_ref)
        z = jnp.dot(x_ref[...], w_ref[...], preferred_element_type=jnp.float32)
        o_ref[...] = z * dis

    return pl.pallas_call(
        body,
        grid=(M // bm,),
        in_specs=[
            pl.BlockSpec((bm, K), lambda i: (i, 0)),
            pl.BlockSpec((K, Cout), lambda i: (0, 0)),
            pl.BlockSpec((2, bm, degp.shape[2]), lambda i: (0, i, 0)),
        ],
        out_specs=pl.BlockSpec((bm, Cout), lambda i: (i, 0)),
        out_shape=jax.ShapeDtypeStruct((M, Cout), jnp.float32),
    )(x_pad, W, degp)


def _tc_mid(part, z1s, degp, b1r, Wcat, bm=1024):
    M, C = z1s.shape

    def body(p_ref, z_ref, degp_ref, b_ref, w_ref, o_ref):
        dis = _dis_block(degp_ref)
        pre = (p_ref[0] + p_ref[1] + z_ref[...]) * dis + b_ref[...]
        h = jnp.maximum(pre, 0.0)
        o_ref[...] = jnp.dot(h, w_ref[...],
                             preferred_element_type=jnp.float32) * dis

    return pl.pallas_call(
        body,
        grid=(M // bm,),
        in_specs=[
            pl.BlockSpec((2, bm, C), lambda i: (0, i, 0)),
            pl.BlockSpec((bm, C), lambda i: (i, 0)),
            pl.BlockSpec((2, bm, degp.shape[2]), lambda i: (0, i, 0)),
            pl.BlockSpec((1, C), lambda i: (0, 0)),
            pl.BlockSpec((C, C), lambda i: (0, 0)),
        ],
        out_specs=pl.BlockSpec((bm, C), lambda i: (i, 0)),
        out_shape=jax.ShapeDtypeStruct((M, C), jnp.float32),
    )(part, z1s, degp, b1r, Wcat)


def _tc_out(part, z2s, degp, bcat, bm=1024):
    M, C = z2s.shape

    def body(p_ref, z_ref, degp_ref, b_ref, o_ref):
        dis = _dis_block(degp_ref)
        o_ref[...] = (p_ref[0] + p_ref[1] + z_ref[...]) * dis + b_ref[...]

    return pl.pallas_call(
        body,
        grid=(M // bm,),
        in_specs=[
            pl.BlockSpec((2, bm, C), lambda i: (0, i, 0)),
            pl.BlockSpec((bm, C), lambda i: (i, 0)),
            pl.BlockSpec((2, bm, degp.shape[2]), lambda i: (0, i, 0)),
            pl.BlockSpec((1, C), lambda i: (0, 0)),
        ],
        out_specs=pl.BlockSpec((bm, C), lambda i: (i, 0)),
        out_shape=jax.ShapeDtypeStruct((M, C), jnp.float32),
    )(part, z2s, degp, bcat)


def kernel(x, edge_index, W1, b1, Wmu, bmu, Wls, bls):
    n, cin = x.shape
    e = edge_index.shape[1]
    src = edge_index[0].astype(jnp.int32)
    dst = edge_index[1].astype(jnp.int32)
    nchunks = -(-e // (NW * CHUNK))
    e_pad = NW * nchunks * CHUNK
    src3 = jnp.concatenate(
        [src, jnp.zeros((e_pad - e,), jnp.int32)]).reshape(NW, nchunks, CHUNK)
    dst3 = jnp.concatenate(
        [dst, jnp.full((e_pad - e,), n, jnp.int32)]).reshape(NW, nchunks, CHUNK)
    x_pad = jnp.pad(x, ((0, N_PAD - n), (0, 0)))
    ones_rows = jnp.ones((CHUNK, cin), jnp.float32)
    zC = jnp.zeros((N_PAD // NS, cin), jnp.float32)
    Wcat = jnp.concatenate([Wmu, Wls], axis=1)
    bcat = jnp.concatenate([bmu, bls]).reshape(1, -1)
    b1r = b1.reshape(1, -1)

    degp = _deg_partials(dst3, ones_rows, zC)
    z1s = _tc_in_scale(x_pad, W1, degp)
    p1 = _agg_partials(z1s, src3, dst3, zC)
    z2s = _tc_mid(p1, z1s, degp, b1r, Wcat)
    p2 = _agg_partials(z2s, src3, dst3, zC)
    out = _tc_out(p2, z2s, degp, bcat)
    ncut = Wcat.shape[1] // 2
    return (out[:n, :ncut], out[:n, ncut:])
